# initial kernel scaffold (unmeasured)
import jax
import jax.numpy as jnp
from jax import lax
from jax.experimental import pallas as pl
from jax.experimental.pallas import tpu as pltpu

NZ = 4
B, H, D, BS = 8, 8, 64, 16
NP_LOCAL = 64
T_LOCAL = NP_LOCAL * BS
NEG = -1e30

_sem_signal = getattr(pl, "semaphore_signal", None) or pltpu.semaphore_signal
_sem_wait = getattr(pl, "semaphore_wait", None) or pltpu.semaphore_wait
_DeviceIdType = getattr(pl, "DeviceIdType", None) or pltpu.DeviceIdType
_CompilerParams = getattr(pltpu, "CompilerParams", None) or getattr(
    pltpu, "TPUCompilerParams"
)


def _body(q_ref, k_ref, v_ref, bt_ref, lens_ref, out_ref,
          comm_ref, send_sems, recv_sems):
    my_x = lax.axis_index("x")
    my_y = lax.axis_index("y")
    my_z = lax.axis_index("z")

    barrier_sem = pltpu.get_barrier_semaphore()
    for d in range(1, NZ):
        _sem_signal(
            barrier_sem, inc=1,
            device_id=(my_x, my_y, (my_z + d) % NZ),
            device_id_type=_DeviceIdType.MESH,
        )
    _sem_wait(barrier_sem, NZ - 1)

    bt = bt_ref[...]
    lens = lens_ref[...]
    base = my_z * NP_LOCAL
    lp = lax.broadcasted_iota(jnp.int32, (B, 64, NP_LOCAL), 2)
    jj = lax.broadcasted_iota(jnp.int32, (B, 64, NP_LOCAL), 1)
    hit = (bt[:, :, None] == base + lp) & (jj < lens[:, :, None])
    cnt = jnp.sum(hit.astype(jnp.float32), axis=1)
    c_tok = jnp.broadcast_to(
        cnt[:, :, None], (B, NP_LOCAL, BS)
    ).reshape(B, T_LOCAL)
    valid = c_tok > 0.0

    kr = k_ref[...].reshape(T_LOCAL, H, D)
    vr = v_ref[...].reshape(T_LOCAL, H, D)
    q = q_ref[...]
    scale = D ** -0.5

    m_cols, l_cols, o_rows = [], [], []
    for h in range(H):
        s = lax.dot_general(
            q[:, h, :], kr[:, h, :],
            (((1,), (1,)), ((), ())),
            preferred_element_type=jnp.float32,
        ) * scale
        s = jnp.where(valid, s, NEG)
        m_h = jnp.max(s, axis=1, keepdims=True)
        p = c_tok * jnp.exp(s - m_h)
        l_h = jnp.sum(p, axis=1, keepdims=True)
        o_h = lax.dot_general(
            p, vr[:, h, :],
            (((1,), (0,)), ((), ())),
            preferred_element_type=jnp.float32,
        )
        m_cols.append(m_h)
        l_cols.append(l_h)
        o_rows.append(o_h[:, None, :])

    m2 = jnp.concatenate(m_cols, axis=1)
    l2 = jnp.concatenate(l_cols, axis=1)
    o2 = jnp.concatenate(o_rows, axis=1).reshape(B * H, D)

    comm_ref[0, 0:B * H, :] = o2
    comm_ref[0, B * H:B * H + 1, :] = m2.reshape(1, B * H)
    comm_ref[0, B * H + 1:B * H + 2, :] = l2.reshape(1, B * H)

    sends = []
    for d in range(1, NZ):
        rdma = pltpu.make_async_remote_copy(
            src_ref=comm_ref.at[0],
            dst_ref=comm_ref.at[d],
            send_sem=send_sems.at[d],
            recv_sem=recv_sems.at[d],
            device_id=(my_x, my_y, (my_z + d) % NZ),
            device_id_type=_DeviceIdType.MESH,
        )
        rdma.start()
        sends.append(rdma)
    for d in range(1, NZ):
        recv = pltpu.make_async_remote_copy(
            src_ref=comm_ref.at[d],
            dst_ref=comm_ref.at[d],
            send_sem=send_sems.at[d],
            recv_sem=recv_sems.at[d],
            device_id=(my_x, my_y, my_z),
            device_id_type=_DeviceIdType.MESH,
        )
        recv.wait_recv()
    for rdma in sends:
        rdma.wait_send()

    allbuf = comm_ref[...]
    o_all = allbuf[:, 0:B * H, :]
    m_all = allbuf[:, B * H, :]
    l_all = allbuf[:, B * H + 1, :]
    gm = jnp.max(m_all, axis=0, keepdims=True)
    sc = jnp.exp(m_all - gm)
    gl = jnp.sum(l_all * sc, axis=0, keepdims=True)
    go = jnp.sum(o_all * sc[:, :, None], axis=0)
    out_ref[...] = go / gl.reshape(B * H, 1)


def kernel(Q, K, V, bt, lens):
    q = Q.reshape(B, H, D)
    lens2 = lens.reshape(B, 1)
    out = pl.pallas_call(
        _body,
        out_shape=jax.ShapeDtypeStruct((B * H, D), jnp.float32),
        in_specs=[pl.BlockSpec(memory_space=pltpu.VMEM)] * 5,
        out_specs=pl.BlockSpec(memory_space=pltpu.VMEM),
        scratch_shapes=[
            pltpu.VMEM((NZ, B * H + 2, D), jnp.float32),
            pltpu.SemaphoreType.DMA((NZ,)),
            pltpu.SemaphoreType.DMA((NZ,)),
        ],
        compiler_params=_CompilerParams(collective_id=0),
    )(q, K, V, bt, lens2)
    return out.reshape(B, 1, H, D)


# baseline (device time: 18909 ns/iter reference)
import jax
import jax.numpy as jnp
from jax import lax
from jax.experimental import pallas as pl
from jax.experimental.pallas import tpu as pltpu

NZ = 4
B, H, D, BS = 8, 8, 64, 16
NP_LOCAL = 64
T_LOCAL = NP_LOCAL * BS
R = B * H
NEG = -1e30

_sem_signal = getattr(pl, "semaphore_signal", None) or pltpu.semaphore_signal
_sem_wait = getattr(pl, "semaphore_wait", None) or pltpu.semaphore_wait
_DeviceIdType = getattr(pl, "DeviceIdType", None) or pltpu.DeviceIdType
_CompilerParams = getattr(pltpu, "CompilerParams", None) or getattr(
    pltpu, "TPUCompilerParams"
)


def _body(q_ref, k_ref, v_ref, bt_ref, lens_ref, out_ref,
          comm_ref, send_sems, recv_sems):
    my_x = lax.axis_index("x")
    my_y = lax.axis_index("y")
    my_z = lax.axis_index("z")

    barrier_sem = pltpu.get_barrier_semaphore()
    for d in range(1, NZ):
        _sem_signal(
            barrier_sem, inc=1,
            device_id=(my_x, my_y, (my_z + d) % NZ),
            device_id_type=_DeviceIdType.MESH,
        )
    _sem_wait(barrier_sem, NZ - 1)

    bt = bt_ref[...]
    lens = lens_ref[...]
    base = my_z * NP_LOCAL
    lp = lax.broadcasted_iota(jnp.int32, (B, 64, NP_LOCAL), 2)
    jj = lax.broadcasted_iota(jnp.int32, (B, 64, NP_LOCAL), 1)
    hit = (bt[:, :, None] == base + lp) & (jj < lens[:, :, None])
    cnt = jnp.sum(hit.astype(jnp.float32), axis=1)
    c_tok = jnp.broadcast_to(
        cnt[:, :, None], (B, NP_LOCAL, BS)
    ).reshape(B, T_LOCAL)
    valid = c_tok > 0.0

    kr = k_ref[...].reshape(T_LOCAL, H, D)
    vr = v_ref[...].reshape(T_LOCAL, H, D)
    q = q_ref[...]
    scale = D ** -0.5

    m_rows, l_rows, o_rows = [], [], []
    for h in range(H):
        s = lax.dot_general(
            q[:, h, :], kr[:, h, :],
            (((1,), (1,)), ((), ())),
            preferred_element_type=jnp.float32,
        ) * scale
        s = jnp.where(valid, s, NEG)
        m_h = jnp.max(s, axis=1, keepdims=True)
        p = c_tok * jnp.exp(s - m_h)
        l_h = jnp.sum(p, axis=1, keepdims=True)
        o_h = lax.dot_general(
            p, vr[:, h, :],
            (((1,), (0,)), ((), ())),
            preferred_element_type=jnp.float32,
        )
        m_rows.append(m_h)
        l_rows.append(l_h)
        o_rows.append(o_h)

    m2 = jnp.concatenate(m_rows, axis=0)
    l2 = jnp.concatenate(l_rows, axis=0)
    o2 = jnp.concatenate(o_rows, axis=0)

    comm_ref[0, :, 0:D] = o2
    comm_ref[0, :, D:D + 1] = m2
    comm_ref[0, :, D + 1:D + 2] = l2

    sends = []
    for d in range(1, NZ):
        rdma = pltpu.make_async_remote_copy(
            src_ref=comm_ref.at[0],
            dst_ref=comm_ref.at[d],
            send_sem=send_sems.at[d],
            recv_sem=recv_sems.at[d],
            device_id=(my_x, my_y, (my_z + d) % NZ),
            device_id_type=_DeviceIdType.MESH,
        )
        rdma.start()
        sends.append(rdma)
    for d in range(1, NZ):
        recv = pltpu.make_async_remote_copy(
            src_ref=comm_ref.at[d],
            dst_ref=comm_ref.at[d],
            send_sem=send_sems.at[d],
            recv_sem=recv_sems.at[d],
            device_id=(my_x, my_y, my_z),
            device_id_type=_DeviceIdType.MESH,
        )
        recv.wait_recv()
    for rdma in sends:
        rdma.wait_send()

    allbuf = comm_ref[...]
    o_all = allbuf[:, :, 0:D]
    m_all = allbuf[:, :, D:D + 1]
    l_all = allbuf[:, :, D + 1:D + 2]
    gm = jnp.max(m_all, axis=0, keepdims=True)
    sc = jnp.exp(m_all - gm)
    gl = jnp.sum(l_all * sc, axis=0)
    go = jnp.sum(o_all * sc, axis=0)
    out_ref[...] = go / gl


def kernel(Q, K, V, bt, lens):
    q = Q.reshape(B, H, D)
    lens2 = lens.reshape(B, 1)
    out = pl.pallas_call(
        _body,
        out_shape=jax.ShapeDtypeStruct((R, D), jnp.float32),
        in_specs=[pl.BlockSpec(memory_space=pltpu.VMEM)] * 5,
        out_specs=pl.BlockSpec(memory_space=pltpu.VMEM),
        scratch_shapes=[
            pltpu.VMEM((NZ, R, D + 2), jnp.float32),
            pltpu.SemaphoreType.DMA((NZ,)),
            pltpu.SemaphoreType.DMA((NZ,)),
        ],
        compiler_params=_CompilerParams(collective_id=0),
    )(q, K, V, bt, lens2)
    return jnp.transpose(out.reshape(H, B, D), (1, 0, 2)).reshape(B, 1, H, D)
